# Initial kernel scaffold; baseline (speedup 1.0000x reference)
import jax
import jax.numpy as jnp
from jax.experimental import pallas as pl


def kernel(codes):
    raise NotImplementedError("write your pallas kernel here")


# speedup vs baseline: 4.9408x; 4.9408x over previous
import jax
import jax.numpy as jnp
from jax import lax
from jax.experimental import pallas as pl
from jax.experimental.pallas import tpu as pltpu
from jax.experimental.pallas import tpu_sc as plsc

_BATCH = 16384
_NSUB = 32
_K = 8192
_NC = 2
_NS = 16
_ROWS = _BATCH // _NS
_CHUNK = 128
_NCHUNK = (_ROWS * 16) // _CHUNK


def _sc_hist(codes_hbm, out_hbm, data_v, idx_v, ones_v, zeros_v, hist_sh):
    c = lax.axis_index("c")
    s = lax.axis_index("s")
    pltpu.sync_copy(codes_hbm.at[pl.ds(s * _ROWS, _ROWS), pl.ds(c * 16, 16)],
                    data_v)

    def fill_ones(i, _):
        ones_v[pl.ds(i * 16, 16)] = jnp.full((16,), 1.0, jnp.float32)
        return 0
    lax.fori_loop(0, _CHUNK // 16, fill_ones, 0)

    def fill_zeros(i, _):
        zeros_v[pl.ds(i * 16, 16)] = jnp.zeros((16,), jnp.float32)
        return 0
    lax.fori_loop(0, _K // 16, fill_zeros, 0)

    pltpu.sync_copy(zeros_v, hist_sh.at[pl.ds(s * _K, _K)])
    plsc.subcore_barrier()

    lane_off = lax.iota(jnp.int32, 16) * _K

    def chunk(j, _):
        def vec(i, _):
            v = data_v[j * (_CHUNK // 16) + i]
            idx_v[j, pl.ds(i * 16, 16)] = v + lane_off
            return 0
        lax.fori_loop(0, _CHUNK // 16, vec, 0, unroll=True)
        pltpu.sync_copy(ones_v, hist_sh.at[idx_v.at[j]], add=True)
        return 0
    lax.fori_loop(0, _NCHUNK, chunk, 0)

    plsc.subcore_barrier()
    pltpu.sync_copy(hist_sh.at[pl.ds(s * _K, _K)], out_hbm.at[c * 16 + s])


_hist_call = pl.kernel(
    _sc_hist,
    out_type=jax.ShapeDtypeStruct((_NSUB, _K), jnp.float32),
    mesh=plsc.VectorSubcoreMesh(core_axis_name="c", subcore_axis_name="s",
                                num_cores=_NC, num_subcores=_NS),
    scratch_types=[
        pltpu.VMEM((_ROWS, 16), jnp.int32),
        pltpu.VMEM((_NCHUNK, _CHUNK), jnp.int32),
        pltpu.VMEM((_CHUNK,), jnp.float32),
        pltpu.VMEM((_K,), jnp.float32),
        pltpu.VMEM_SHARED((_NS * _K,), jnp.float32),
    ],
    compiler_params=pltpu.CompilerParams(use_tc_tiling_on_sc=False),
)


def _tc_entropy(hist_ref, out_ref):
    counts = hist_ref[...]
    probs = counts * (1.0 / _BATCH) + 1e-8
    z = jnp.sum(probs, axis=1, keepdims=True)
    p = probs / z
    ent = -jnp.sum(p * jnp.log(p), axis=1, keepdims=True)
    target = jnp.log(jnp.float32(_K))
    d = target - ent
    out_ref[0, 0] = jnp.sum(d * d) * (1.0 / _NSUB)


def kernel(codes):
    hist = _hist_call(codes)
    loss = pl.pallas_call(
        _tc_entropy,
        out_shape=jax.ShapeDtypeStruct((1, 1), jnp.float32),
        out_specs=pl.BlockSpec(memory_space=pltpu.SMEM),
    )(hist)
    return loss[0, 0]


# speedup vs baseline: 6.1722x; 1.2492x over previous
import jax
import jax.numpy as jnp
from jax import lax
from jax.experimental import pallas as pl
from jax.experimental.pallas import tpu as pltpu
from jax.experimental.pallas import tpu_sc as plsc

_BATCH = 16384
_NSUB = 32
_K = 8192
_NC = 2
_NS = 16
_ROWS = _BATCH // _NS
_CHUNK = 128
_NCHUNK = (_ROWS * 16) // _CHUNK


_PIPE = 8


def _sc_hist(codes_hbm, out_hbm, data_v, idx_v, ones_v, zeros_v, hist_sh,
             stage_sem, sem):
    c = lax.axis_index("c")
    s = lax.axis_index("s")
    stage = pltpu.async_copy(
        codes_hbm.at[pl.ds(s * _ROWS, _ROWS), pl.ds(c * 16, 16)], data_v,
        stage_sem)

    def fill_ones(i, _):
        ones_v[pl.ds(i * 16, 16)] = jnp.full((16,), 1.0, jnp.float32)
        return 0
    lax.fori_loop(0, _CHUNK // 16, fill_ones, 0)

    def fill_zeros(i, _):
        zeros_v[pl.ds(i * 16, 16)] = jnp.zeros((16,), jnp.float32)
        return 0
    lax.fori_loop(0, _K // 16, fill_zeros, 0)

    pltpu.sync_copy(zeros_v, hist_sh.at[pl.ds(s * _K, _K)])
    stage.wait()
    plsc.subcore_barrier()

    lane_off = lax.iota(jnp.int32, 16) * _K

    def compute_and_fire(j):
        def vec(i, _):
            v = data_v[j * (_CHUNK // 16) + i]
            idx_v[j, pl.ds(i * 16, 16)] = v + lane_off
            return 0
        lax.fori_loop(0, _CHUNK // 16, vec, 0, unroll=True)
        pltpu.async_copy(ones_v, hist_sh.at[idx_v.at[j]], sem, add=True)

    def head(j, _):
        compute_and_fire(j)
        return 0
    lax.fori_loop(0, _PIPE, head, 0)

    def body(j, _):
        pltpu.make_async_copy(ones_v, hist_sh.at[idx_v.at[j - _PIPE]],
                              sem).wait()
        compute_and_fire(j)
        return 0
    lax.fori_loop(_PIPE, _NCHUNK, body, 0)

    def drain(j, _):
        pltpu.make_async_copy(ones_v, hist_sh.at[idx_v.at[j]], sem).wait()
        return 0
    lax.fori_loop(_NCHUNK - _PIPE, _NCHUNK, drain, 0)

    plsc.subcore_barrier()
    pltpu.sync_copy(hist_sh.at[pl.ds(s * _K, _K)], out_hbm.at[c * 16 + s])


_hist_call = pl.kernel(
    _sc_hist,
    out_type=jax.ShapeDtypeStruct((_NSUB, _K), jnp.float32),
    mesh=plsc.VectorSubcoreMesh(core_axis_name="c", subcore_axis_name="s",
                                num_cores=_NC, num_subcores=_NS),
    scratch_types=[
        pltpu.VMEM((_ROWS, 16), jnp.int32),
        pltpu.VMEM((_NCHUNK, _CHUNK), jnp.int32),
        pltpu.VMEM((_CHUNK,), jnp.float32),
        pltpu.VMEM((_K,), jnp.float32),
        pltpu.VMEM_SHARED((_NS * _K,), jnp.float32),
        pltpu.SemaphoreType.DMA,
        pltpu.SemaphoreType.DMA,
    ],
    compiler_params=pltpu.CompilerParams(use_tc_tiling_on_sc=False),
)


def _tc_entropy(hist_ref, out_ref):
    counts = hist_ref[...]
    probs = counts * (1.0 / _BATCH) + 1e-8
    z = jnp.sum(probs, axis=1, keepdims=True)
    p = probs / z
    ent = -jnp.sum(p * jnp.log(p), axis=1, keepdims=True)
    target = jnp.log(jnp.float32(_K))
    d = target - ent
    out_ref[0, 0] = jnp.sum(d * d) * (1.0 / _NSUB)


def kernel(codes):
    hist = _hist_call(codes)
    loss = pl.pallas_call(
        _tc_entropy,
        out_shape=jax.ShapeDtypeStruct((1, 1), jnp.float32),
        out_specs=pl.BlockSpec(memory_space=pltpu.SMEM),
    )(hist)
    return loss[0, 0]
